# Initial kernel scaffold; baseline (speedup 1.0000x reference)
import jax
import jax.numpy as jnp
from jax.experimental import pallas as pl


def kernel(x, edge_index, batch, params):
    src, dst = edge_index[0], edge_index[1]
    N = x.shape[0]
    G = 128
    T = 5.0

    h = x
    for p in params["causal"]:
        hW1 = h @ p["W1"]
        agg = jnp.zeros_like(hW1).at[dst].add(hW1[src])
        u = (1.0 + p["eps"]) * hW1 + agg + p["b1"]
        h = jax.nn.relu(jax.nn.relu(u) @ p["W2"] + p["b2"])
    Z = h

    nm = jnp.stack([jax.nn.sigmoid((jax.nn.relu(Z @ params["node_masks"][k]["W1"] + params["node_masks"][k]["b1"]) @ params["node_masks"][k]["W2"] + params["node_masks"][k]["b2"]) / T) for k in range(4)], 1)
    fm = jnp.stack([jax.nn.sigmoid((jax.nn.relu(Z @ params["feat_masks"][k]["W1"] + params["feat_masks"][k]["b1"]) @ params["feat_masks"][k]["W2"] + params["feat_masks"][k]["b2"]) / T) for k in range(4)], 1)

    efs, efd = Z[src], Z[dst]
    ems = []
    for k in range(4):
        p = params["edge_masks"][k]
        W1a, W1b = p["W1"][:64], p["W1"][64:]
        hmid = jax.nn.relu(efs @ W1a + efd @ W1b + p["b1"])
        ems.append(jax.nn.sigmoid((hmid @ p["W2"] + p["b2"]) / T))
    em = jnp.stack(ems, 1)
    ew = em[:, :, 0]

    mx = x[:, None, :] * nm * fm
    h4 = mx
    for p in params["classifier"]:
        hW1 = jnp.einsum('nkd,dh->nkh', h4, p["W1"])
        msg = hW1[src] * ew[:, :, None]
        agg = jnp.zeros_like(hW1).at[dst].add(msg)
        u = (1.0 + p["eps"]) * hW1 + agg + p["b1"]
        h4 = jax.nn.relu(jax.nn.relu(u) @ p["W2"] + p["b2"])
    mZ = h4

    cnt = jnp.maximum(jnp.zeros((G,), jnp.float32).at[batch].add(1.0), 1.0)
    h_orig = (jnp.zeros((G, 64), jnp.float32).at[batch].add(Z)) / cnt[:, None]
    hs = (jnp.zeros((G, 4, 64), jnp.float32).at[batch].add(mZ)) / cnt[:, None, None]
    W = jnp.stack([params["classifiers"][k]["W"] for k in range(4)], 0)
    b = jnp.stack([params["classifiers"][k]["b"] for k in range(4)], 0)
    lg = jnp.einsum('gkh,kho->gko', hs, W) + b
    return (lg, hs, h_orig, nm, em, fm)


# speedup vs baseline: 1.0001x; 1.0001x over previous
import jax
import jax.numpy as jnp
from jax.experimental import pallas as pl


def _mlp_a(p, x):
    return jax.nn.relu(x @ p["W1"] + p["b1"]) @ p["W2"] + p["b2"]


def _gin_a(layers, x, edge_index, edge_weight=None):
    src = edge_index[0]
    dst = edge_index[1]
    h = x
    for p in layers:
        msg = h[src]
        if edge_weight is not None:
            msg = msg * edge_weight[:, None]
        agg = jnp.zeros_like(h).at[dst].add(msg)
        h = jax.nn.relu(_mlp_a(p, (1.0 + p["eps"]) * h + agg))
    return h


def _pool(h, batch):
    G = 128
    s = jax.ops.segment_sum(h, batch, num_segments=G)
    c = jax.ops.segment_sum(jnp.ones((h.shape[0],), jnp.float32), batch, num_segments=G)
    return s / jnp.maximum(c, 1.0)[:, None]


def _hc(logits):
    return jax.nn.sigmoid(logits / 5.0)


def kernel(x, edge_index, batch, params):
    Z = _gin_a(params["causal"], x, edge_index)
    src = edge_index[0]
    dst = edge_index[1]
    edge_feat = jnp.concatenate([Z[src], Z[dst]], axis=1)
    node_ms, edge_ms, feat_ms, logits, h_st = [], [], [], [], []
    for k in range(4):
        nm = _hc(_mlp_a(params["node_masks"][k], Z))
        em = _hc(_mlp_a(params["edge_masks"][k], edge_feat))
        fm = _hc(_mlp_a(params["feat_masks"][k], Z))
        masked_x = x * nm * fm
        ew = em[:, 0]
        mZ = _gin_a(params["classifier"], masked_x, edge_index, ew)
        hs = _pool(mZ, batch)
        lg = hs @ params["classifiers"][k]["W"] + params["classifiers"][k]["b"]
        node_ms.append(nm)
        edge_ms.append(em)
        feat_ms.append(fm)
        logits.append(lg)
        h_st.append(hs)
    return (jnp.stack(logits, axis=1), jnp.stack(h_st, axis=1), _pool(Z, batch),
            jnp.stack(node_ms, axis=1), jnp.stack(edge_ms, axis=1), jnp.stack(feat_ms, axis=1))


# speedup vs baseline: 5.4437x; 5.4434x over previous
import functools

import jax
import jax.numpy as jnp
from jax import lax
from jax.experimental import pallas as pl
from jax.experimental.pallas import tpu as pltpu
from jax.experimental.pallas import tpu_sc as plsc

N = 10000
NPAD = 10240
E = 320000
G = 128
TEMP = 5.0
CH = 100
NCHUNK = E // CH
NB = 512
NGRID = NPAD // NB
EB = 2000
ZR = NPAD // 16

_f32 = jnp.float32
_SDS = jax.ShapeDtypeStruct


def _mesh():
    return plsc.VectorSubcoreMesh(core_axis_name="c", subcore_axis_name="s")


def _mm_body(x_ref, w_ref, o_ref):
    o_ref[...] = jnp.dot(x_ref[...], w_ref[...], preferred_element_type=_f32)


def tc_matmul(x, w):
    n, din = x.shape
    dout = w.shape[1]
    return pl.pallas_call(
        _mm_body,
        grid=(n // NB,),
        in_specs=[pl.BlockSpec((NB, din), lambda i: (i, 0)),
                  pl.BlockSpec((din, dout), lambda i: (0, 0))],
        out_specs=pl.BlockSpec((NB, dout), lambda i: (i, 0)),
        out_shape=_SDS((n, dout), _f32),
    )(x, w)


def _gin_layer_body(last, hw_ref, p0_ref, p1_ref, w2_ref, b1_ref, b2_ref,
                    eps_ref, w1n_ref, *out_refs):
    u = hw_ref[...] * eps_ref[0, 0] + p0_ref[...] + p1_ref[...] + b1_ref[...]
    v = jnp.maximum(u, 0.0)
    h = jnp.maximum(jnp.dot(v, w2_ref[...], preferred_element_type=_f32)
                    + b2_ref[...], 0.0)
    if last:
        out_refs[0][...] = h
    else:
        out_refs[0][...] = h
        out_refs[1][...] = jnp.dot(h, w1n_ref[...], preferred_element_type=_f32)


def tc_gin_layer(hW, p0, p1, p, w1_next):
    last = w1_next is None
    wspec = lambda shp: pl.BlockSpec(shp, lambda i: (0, 0))
    in_specs = [pl.BlockSpec((NB, 64), lambda i: (i, 0))] * 3 + [
        wspec((64, 64)), wspec((1, 64)), wspec((1, 64)),
        pl.BlockSpec(memory_space=pltpu.SMEM), wspec((64, 64))]
    out_specs = [pl.BlockSpec((NB, 64), lambda i: (i, 0))] * (1 if last else 2)
    out_shape = [_SDS((NPAD, 64), _f32)] * (1 if last else 2)
    epsp = (1.0 + p["eps"]).reshape(1, 1).astype(_f32)
    w1n = jnp.zeros((64, 64), _f32) if last else w1_next
    outs = pl.pallas_call(
        functools.partial(_gin_layer_body, last),
        grid=(NGRID,),
        in_specs=in_specs,
        out_specs=out_specs,
        out_shape=out_shape,
    )(hW, p0, p1, p["W2"], p["b1"].reshape(1, 64), p["b2"].reshape(1, 64),
      epsp, w1n)
    return outs[0] if last else outs


def _masks_body(x_ref, z_ref, nw1_ref, nb1_ref, nw2_ref, nb2_ref,
                fw1_ref, fb1_ref, fw2_0, fw2_1, fw2_2, fw2_3,
                fb2_0, fb2_1, fb2_2, fb2_3, w1c_ref,
                nm_ref, fm_ref, t0_ref, t1_ref, t2_ref, t3_ref):
    z = z_ref[...]
    x = x_ref[...]
    h1 = jnp.maximum(jnp.dot(z, nw1_ref[...], preferred_element_type=_f32)
                     + nb1_ref[...], 0.0)
    t = h1 * nw2_ref[...]
    nm_parts = [jnp.sum(t[:, k * 64:(k + 1) * 64], axis=1, keepdims=True)
                for k in range(4)]
    nm = jax.nn.sigmoid((jnp.concatenate(nm_parts, axis=1) + nb2_ref[...])
                        / TEMP)
    nm_ref[...] = nm
    h2 = jnp.maximum(jnp.dot(z, fw1_ref[...], preferred_element_type=_f32)
                     + fb1_ref[...], 0.0)
    fw2 = (fw2_0, fw2_1, fw2_2, fw2_3)
    fb2 = (fb2_0, fb2_1, fb2_2, fb2_3)
    fms = []
    m1s = []
    for k in range(4):
        fmk = jax.nn.sigmoid(
            (jnp.dot(h2[:, k * 64:(k + 1) * 64], fw2[k][...],
                     preferred_element_type=_f32) + fb2[k][...]) / TEMP)
        fms.append(fmk)
        mx = x * nm[:, k:k + 1] * fmk
        m1s.append(jnp.dot(mx, w1c_ref[...], preferred_element_type=_f32))
    fm_ref[...] = jnp.concatenate(fms, axis=1)
    t0_ref[...] = m1s[0]
    t1_ref[...] = m1s[1]
    t2_ref[...] = m1s[2]
    t3_ref[...] = m1s[3]


def tc_masks(x_pad, Z, params):
    nm_p = params["node_masks"]
    fm_p = params["feat_masks"]
    w1c = params["classifier"][0]["W1"]
    nw1 = jnp.concatenate([p["W1"] for p in nm_p], axis=1)
    nb1 = jnp.concatenate([p["b1"] for p in nm_p]).reshape(1, 256)
    nw2 = jnp.concatenate([p["W2"][:, 0] for p in nm_p]).reshape(1, 256)
    nb2 = jnp.stack([p["b2"][0] for p in nm_p]).reshape(1, 4)
    fw1 = jnp.concatenate([p["W1"] for p in fm_p], axis=1)
    fb1 = jnp.concatenate([p["b1"] for p in fm_p]).reshape(1, 256)
    wspec = lambda shp: pl.BlockSpec(shp, lambda i: (0, 0))
    in_specs = [pl.BlockSpec((NB, 128), lambda i: (i, 0)),
                pl.BlockSpec((NB, 64), lambda i: (i, 0)),
                wspec((64, 256)), wspec((1, 256)), wspec((1, 256)),
                wspec((1, 4)), wspec((64, 256)), wspec((1, 256)),
                wspec((64, 128)), wspec((64, 128)), wspec((64, 128)),
                wspec((64, 128)), wspec((1, 128)), wspec((1, 128)),
                wspec((1, 128)), wspec((1, 128)), wspec((128, 64))]
    out_specs = [pl.BlockSpec((NB, 4), lambda i: (i, 0)),
                 pl.BlockSpec((NB, 512), lambda i: (i, 0))] + [
                 pl.BlockSpec((NB, 64), lambda i: (i, 0))] * 4
    out_shape = [_SDS((NPAD, 4), _f32), _SDS((NPAD, 512), _f32)] + [
                 _SDS((NPAD, 64), _f32)] * 4
    args = [x_pad, Z, nw1, nb1, nw2, nb2, fw1, fb1]
    args += [fm_p[k]["W2"] for k in range(4)]
    args += [fm_p[k]["b2"].reshape(1, 128) for k in range(4)]
    args += [w1c]
    return pl.pallas_call(
        _masks_body, grid=(NGRID,), in_specs=in_specs, out_specs=out_specs,
        out_shape=out_shape)(*args)


def _em_body(efs_ref, efd_ref, wa_ref, wb_ref, b1_ref, w2_ref, b2_ref, em_ref):
    h = jnp.maximum(
        jnp.dot(efs_ref[...], wa_ref[...], preferred_element_type=_f32)
        + jnp.dot(efd_ref[...], wb_ref[...], preferred_element_type=_f32)
        + b1_ref[...], 0.0)
    t = h * w2_ref[...]
    parts = [jnp.sum(t[:, k * 64:(k + 1) * 64], axis=1, keepdims=True)
             for k in range(4)]
    em_ref[...] = jax.nn.sigmoid(
        (jnp.concatenate(parts, axis=1) + b2_ref[...]) / TEMP)


def tc_em(efs, efd, params):
    em_p = params["edge_masks"]
    wa = jnp.concatenate([p["W1"][:64] for p in em_p], axis=1)
    wb = jnp.concatenate([p["W1"][64:] for p in em_p], axis=1)
    b1 = jnp.concatenate([p["b1"] for p in em_p]).reshape(1, 256)
    w2 = jnp.concatenate([p["W2"][:, 0] for p in em_p]).reshape(1, 256)
    b2 = jnp.stack([p["b2"][0] for p in em_p]).reshape(1, 4)
    wspec = lambda shp: pl.BlockSpec(shp, lambda i: (0, 0))
    return pl.pallas_call(
        _em_body,
        grid=(E // EB,),
        in_specs=[pl.BlockSpec((EB, 64), lambda i: (i, 0)),
                  pl.BlockSpec((EB, 64), lambda i: (i, 0)),
                  wspec((64, 256)), wspec((64, 256)), wspec((1, 256)),
                  wspec((1, 256)), wspec((1, 4))],
        out_specs=pl.BlockSpec((EB, 4), lambda i: (i, 0)),
        out_shape=_SDS((E, 4), _f32),
    )(efs, efd, wa, wb, b1, w2, b2)


def _cls_layer_body(last, t0_ref, t1_ref, t2_ref, t3_ref,
                    u0_ref, u1_ref, u2_ref, u3_ref, w2_ref, b1_ref,
                    b2_ref, eps_ref, w1n_ref, *out_refs):
    w2 = w2_ref[...]
    b2 = b2_ref[...]
    t_refs = (t0_ref, t1_ref, t2_ref, t3_ref)
    u_refs = (u0_ref, u1_ref, u2_ref, u3_ref)
    hs = []
    for k in range(4):
        u = t_refs[k][...] * eps_ref[0, 0] + u_refs[k][...] + b1_ref[...]
        v = jnp.maximum(u, 0.0)
        hs.append(jnp.maximum(jnp.dot(v, w2, preferred_element_type=_f32)
                              + b2, 0.0))
    if last:
        out_refs[0][...] = jnp.concatenate(hs, axis=1)
    else:
        w1n = w1n_ref[...]
        for k in range(4):
            out_refs[k][...] = jnp.dot(hs[k], w1n,
                                       preferred_element_type=_f32)


def tc_cls_layer(Ts, Us, p, w1_next):
    last = w1_next is None
    wspec = lambda shp: pl.BlockSpec(shp, lambda i: (0, 0))
    in_specs = [pl.BlockSpec((NB, 64), lambda i: (i, 0))] * 8 + [
        wspec((64, 64)), wspec((1, 64)), wspec((1, 64)),
        pl.BlockSpec(memory_space=pltpu.SMEM), wspec((64, 64))]
    if last:
        out_specs = [pl.BlockSpec((NB, 256), lambda i: (i, 0))]
        out_shape = [_SDS((NPAD, 256), _f32)]
    else:
        out_specs = [pl.BlockSpec((NB, 64), lambda i: (i, 0))] * 4
        out_shape = [_SDS((NPAD, 64), _f32)] * 4
    epsp = (1.0 + p["eps"]).reshape(1, 1).astype(_f32)
    w1n = jnp.zeros((64, 64), _f32) if last else w1_next
    outs = pl.pallas_call(
        functools.partial(_cls_layer_body, last),
        grid=(NGRID,),
        in_specs=in_specs,
        out_specs=out_specs,
        out_shape=out_shape,
    )(*Ts, *Us, p["W2"], p["b1"].reshape(1, 64), p["b2"].reshape(1, 64),
      epsp, w1n)
    return outs[0] if last else outs


def _pool_body(z_ref, mz_ref, b_ref, wl_ref, bl_ref,
               ho_ref, hs_ref, lg_ref, sz_acc, sm_acc, cnt_acc):
    i = pl.program_id(0)

    @pl.when(i == 0)
    def _init():
        sz_acc[...] = jnp.zeros_like(sz_acc)
        sm_acc[...] = jnp.zeros_like(sm_acc)
        cnt_acc[...] = jnp.zeros_like(cnt_acc)

    bcol = jnp.reshape(b_ref[...], (NB, 1))
    iota = lax.broadcasted_iota(jnp.int32, (NB, G), 1)
    rows = i * NB + lax.broadcasted_iota(jnp.int32, (NB, 1), 0)
    valid = (rows < N).astype(_f32)
    oh = (bcol == iota).astype(_f32) * valid
    dn = (((0,), (0,)), ((), ()))
    sz_acc[...] += lax.dot_general(oh, z_ref[...], dn,
                                   preferred_element_type=_f32)
    sm_acc[...] += lax.dot_general(oh, mz_ref[...], dn,
                                   preferred_element_type=_f32)
    cnt_acc[...] += lax.dot_general(oh, valid, dn,
                                    preferred_element_type=_f32)

    @pl.when(i == NGRID - 1)
    def _fin():
        cnt = jnp.maximum(cnt_acc[...], 1.0)
        ho = sz_acc[...] / cnt
        hs = sm_acc[...] / cnt
        ho_ref[...] = ho
        hs_ref[...] = hs
        wl = wl_ref[...]
        bl = bl_ref[...]
        lgs = [jnp.dot(hs[:, k * 64:(k + 1) * 64], wl[:, k * 10:(k + 1) * 10],
                       preferred_element_type=_f32) + bl[:, k * 10:(k + 1) * 10]
               for k in range(4)]
        lg_ref[...] = jnp.concatenate(lgs, axis=1)


def tc_pool(Z, mZ, batch3, params):
    wl = jnp.concatenate([params["classifiers"][k]["W"] for k in range(4)],
                         axis=1)
    bl = jnp.concatenate([params["classifiers"][k]["b"] for k in range(4)]
                         ).reshape(1, 40)
    wspec = lambda shp: pl.BlockSpec(shp, lambda i: (0, 0))
    return pl.pallas_call(
        _pool_body,
        grid=(NGRID,),
        in_specs=[pl.BlockSpec((NB, 64), lambda i: (i, 0)),
                  pl.BlockSpec((NB, 256), lambda i: (i, 0)),
                  pl.BlockSpec((1, NB, 1), lambda i: (i, 0, 0)),
                  wspec((64, 40)), wspec((1, 40))],
        out_specs=[pl.BlockSpec((G, 64), lambda i: (0, 0)),
                   pl.BlockSpec((G, 256), lambda i: (0, 0)),
                   pl.BlockSpec((G, 40), lambda i: (0, 0))],
        out_shape=[_SDS((G, 64), _f32), _SDS((G, 256), _f32),
                   _SDS((G, 40), _f32)],
        scratch_shapes=[pltpu.VMEM((G, 64), _f32), pltpu.VMEM((G, 256), _f32),
                        pltpu.VMEM((G, 1), _f32)],
    )(Z, mZ, batch3, wl, bl)


_GDN = lax.GatherDimensionNumbers(offset_dims=(), collapsed_slice_dims=(0,),
                                  start_index_map=(0,))


def _lane_bcast(w, idx):
    return lax.gather(w, idx[:, None], _GDN, (1,),
                      mode=lax.GatherScatterMode.PROMISE_IN_BOUNDS)

def _make_sc_causal():
    cpw = NCHUNK // 32

    def body(tbl, srcr, dstr, zr, out0, out1,
             src_v, dst_v, rows0, rows1, accum, sem0, sem1):
        c = lax.axis_index("c")
        s = lax.axis_index("s")
        wid = s * 2 + c
        pltpu.sync_copy(zr.at[pl.ds(s * ZR, ZR)], accum.at[pl.ds(s * ZR, ZR)])
        pltpu.sync_copy(srcr.at[wid], src_v)
        pltpu.sync_copy(dstr.at[wid], dst_v)
        plsc.subcore_barrier()

        def g_start(j, buf, sem):
            pltpu.async_copy(tbl.at[src_v.at[j]], buf, sem)

        def g_wait(buf, sem):
            pltpu.make_async_copy(tbl.at[src_v.at[0]], buf, sem).wait()

        g_start(0, rows0, sem0)
        g_start(1, rows1, sem1)

        def pair(i, carry):
            j0 = i * 2
            g_wait(rows0, sem0)
            pltpu.sync_copy(rows0, accum.at[dst_v.at[j0]], add=True)

            @pl.when(j0 + 2 < cpw)
            def _():
                g_start(j0 + 2, rows0, sem0)

            g_wait(rows1, sem1)
            pltpu.sync_copy(rows1, accum.at[dst_v.at[j0 + 1]], add=True)

            @pl.when(j0 + 3 < cpw)
            def _():
                g_start(j0 + 3, rows1, sem1)
            return carry

        lax.fori_loop(0, cpw // 2, pair, 0)
        plsc.subcore_barrier()
        sl = pl.ds(s * ZR, ZR)

        @pl.when(c == 0)
        def _():
            pltpu.sync_copy(accum.at[sl], out0.at[sl])

        @pl.when(c == 1)
        def _():
            pltpu.sync_copy(accum.at[sl], out1.at[sl])

    return pl.kernel(
        body,
        out_type=(_SDS((NPAD, 64), _f32), _SDS((NPAD, 64), _f32)),
        mesh=_mesh(),
        compiler_params=pltpu.CompilerParams(use_tc_tiling_on_sc=False),
        scratch_types=[pltpu.VMEM((NCHUNK // 32, CH), jnp.int32),
                       pltpu.VMEM((NCHUNK // 32, CH), jnp.int32),
                       pltpu.VMEM((CH, 64), _f32),
                       pltpu.VMEM((CH, 64), _f32),
                       pltpu.VMEM_SHARED((NPAD, 64), _f32),
                       pltpu.SemaphoreType.DMA, pltpu.SemaphoreType.DMA],
    )


def _make_sc_ef():
    cpw = NCHUNK // 32

    def body(tbl, srcr, dstr, efs, efd,
             src_v, dst_v, bs0, bs1, bd0, bd1, sem0, sem1):
        c = lax.axis_index("c")
        s = lax.axis_index("s")
        wid = s * 2 + c
        pltpu.sync_copy(srcr.at[wid], src_v)
        pltpu.sync_copy(dstr.at[wid], dst_v)
        cbase = wid * cpw

        def g_start(j, bs, bd, sem):
            pltpu.async_copy(tbl.at[src_v.at[j]], bs, sem)
            pltpu.async_copy(tbl.at[dst_v.at[j]], bd, sem)

        def g_wait(bs, bd, sem):
            pltpu.make_async_copy(tbl.at[src_v.at[0]], bs, sem).wait()
            pltpu.make_async_copy(tbl.at[src_v.at[0]], bd, sem).wait()

        g_start(0, bs0, bd0, sem0)
        g_start(1, bs1, bd1, sem1)

        def pair(i, carry):
            j0 = i * 2
            g_wait(bs0, bd0, sem0)
            pltpu.sync_copy(bs0, efs.at[cbase + j0])
            pltpu.sync_copy(bd0, efd.at[cbase + j0])

            @pl.when(j0 + 2 < cpw)
            def _():
                g_start(j0 + 2, bs0, bd0, sem0)

            g_wait(bs1, bd1, sem1)
            pltpu.sync_copy(bs1, efs.at[cbase + j0 + 1])
            pltpu.sync_copy(bd1, efd.at[cbase + j0 + 1])

            @pl.when(j0 + 3 < cpw)
            def _():
                g_start(j0 + 3, bs1, bd1, sem1)
            return carry

        lax.fori_loop(0, cpw // 2, pair, 0)

    return pl.kernel(
        body,
        out_type=(_SDS((NCHUNK, CH, 64), _f32), _SDS((NCHUNK, CH, 64), _f32)),
        mesh=_mesh(),
        compiler_params=pltpu.CompilerParams(use_tc_tiling_on_sc=False),
        scratch_types=[pltpu.VMEM((NCHUNK // 32, CH), jnp.int32),
                       pltpu.VMEM((NCHUNK // 32, CH), jnp.int32),
                       pltpu.VMEM((CH, 64), _f32), pltpu.VMEM((CH, 64), _f32),
                       pltpu.VMEM((CH, 64), _f32), pltpu.VMEM((CH, 64), _f32),
                       pltpu.SemaphoreType.DMA, pltpu.SemaphoreType.DMA],
    )


def _make_sc_cls(kbase):
    cpt = NCHUNK // 16
    wrows = CH // 4

    def body(tab, srcr, dstr, ewr, zr, uab,
             src_v, dst_v, rows0, rows1, ew0, ew1, accum,
             sem0, sem1, semw0, semw1):
        c = lax.axis_index("c")
        s = lax.axis_index("s")
        pltpu.sync_copy(zr.at[pl.ds(s * ZR, ZR)], accum.at[pl.ds(s * ZR, ZR)])
        pltpu.sync_copy(srcr.at[s], src_v)
        pltpu.sync_copy(dstr.at[s], dst_v)
        plsc.subcore_barrier()
        tc_ = tab.at[c]

        def g_start(j, buf, ewb, sem, semw):
            pltpu.async_copy(tc_.at[src_v.at[j]], buf, sem)
            pltpu.async_copy(ewr.at[s * cpt + j], ewb, semw)

        def g_wait(buf, ewb, sem, semw):
            pltpu.make_async_copy(tc_.at[src_v.at[0]], buf, sem).wait()
            pltpu.make_async_copy(ewr.at[0], ewb, semw).wait()

        def scale_k(buf, ewb, kidx):
            def grp(g, carry):
                w = ewb[g]
                for i in range(4):
                    r = g * 4 + i
                    wk = _lane_bcast(w, jnp.full((16,), 4 * i + kidx,
                                                 jnp.int32))
                    for q in range(4):
                        sl = pl.ds(q * 16, 16)
                        buf[r, sl] = buf[r, sl] * wk
                return carry

            lax.fori_loop(0, wrows, grp, 0)

        def scale(buf, ewb):
            @pl.when(c == 0)
            def _():
                scale_k(buf, ewb, kbase)

            @pl.when(c == 1)
            def _():
                scale_k(buf, ewb, kbase + 1)

        g_start(0, rows0, ew0, sem0, semw0)
        g_start(1, rows1, ew1, sem1, semw1)

        def pair(i, carry):
            j0 = i * 2
            g_wait(rows0, ew0, sem0, semw0)
            scale(rows0, ew0)
            pltpu.sync_copy(rows0, accum.at[dst_v.at[j0]], add=True)

            @pl.when(j0 + 2 < cpt)
            def _():
                g_start(j0 + 2, rows0, ew0, sem0, semw0)

            g_wait(rows1, ew1, sem1, semw1)
            scale(rows1, ew1)
            pltpu.sync_copy(rows1, accum.at[dst_v.at[j0 + 1]], add=True)

            @pl.when(j0 + 3 < cpt)
            def _():
                g_start(j0 + 3, rows1, ew1, sem1, semw1)
            return carry

        lax.fori_loop(0, cpt // 2, pair, 0)
        plsc.subcore_barrier()
        sl = pl.ds(s * ZR, ZR)
        pltpu.sync_copy(accum.at[sl], uab.at[c, sl])

    return pl.kernel(
        body,
        out_type=_SDS((2, NPAD, 64), _f32),
        mesh=_mesh(),
        compiler_params=pltpu.CompilerParams(use_tc_tiling_on_sc=False),
        scratch_types=[pltpu.VMEM((NCHUNK // 16, CH), jnp.int32),
                       pltpu.VMEM((NCHUNK // 16, CH), jnp.int32),
                       pltpu.VMEM((CH, 64), _f32),
                       pltpu.VMEM((CH, 64), _f32),
                       pltpu.VMEM((CH // 4, 16), _f32),
                       pltpu.VMEM((CH // 4, 16), _f32),
                       pltpu.VMEM_SHARED((NPAD, 64), _f32),
                       pltpu.SemaphoreType.DMA, pltpu.SemaphoreType.DMA,
                       pltpu.SemaphoreType.DMA, pltpu.SemaphoreType.DMA],
    )


def kernel(x, edge_index, batch, params):
    src32 = edge_index[0].reshape(32, NCHUNK // 32, CH)
    dst32 = edge_index[1].reshape(32, NCHUNK // 32, CH)
    src16 = edge_index[0].reshape(16, NCHUNK // 16, CH)
    dst16 = edge_index[1].reshape(16, NCHUNK // 16, CH)
    x_pad = jnp.pad(x, ((0, NPAD - N), (0, 0)))
    batch3 = jnp.pad(batch, (0, NPAD - N)).astype(jnp.int32).reshape(
        NGRID, NB, 1)
    zeros64 = jnp.zeros((NPAD, 64), _f32)
    zeros128 = jnp.zeros((NPAD, 128), _f32)

    sc_causal = _make_sc_causal()
    sc_ef = _make_sc_ef()
    sc_cls01 = _make_sc_cls(0)
    sc_cls23 = _make_sc_cls(2)

    cl = params["causal"]
    hW = tc_matmul(x_pad, cl[0]["W1"])
    Z = None
    for li in range(3):
        p0, p1 = sc_causal(hW, src32, dst32, zeros64)
        if li < 2:
            _, hW = tc_gin_layer(hW, p0, p1, cl[li], cl[li + 1]["W1"])
        else:
            Z = tc_gin_layer(hW, p0, p1, cl[li], None)

    nm4, fm4, T0, T1, T2, T3 = tc_masks(x_pad, Z, params)
    efs, efd = sc_ef(Z, src32, dst32)
    em4 = tc_em(efs.reshape(E, 64), efd.reshape(E, 64), params)
    ew3 = em4.reshape(NCHUNK, CH // 4, 16)

    clf = params["classifier"]
    Ts = [T0, T1, T2, T3]
    mZ = None
    for li in range(3):
        Uab = sc_cls01(jnp.stack([Ts[0], Ts[1]]), src16, dst16, ew3, zeros64)
        Ucd = sc_cls23(jnp.stack([Ts[2], Ts[3]]), src16, dst16, ew3, zeros64)
        Us = [Uab[0], Uab[1], Ucd[0], Ucd[1]]
        if li < 2:
            Ts = tc_cls_layer(Ts, Us, clf[li], clf[li + 1]["W1"])
        else:
            mZ = tc_cls_layer(Ts, Us, clf[li], None)

    h_orig, hs_flat, lg_flat = tc_pool(Z, mZ, batch3, params)

    return (lg_flat.reshape(G, 4, 10),
            hs_flat.reshape(G, 4, 64),
            h_orig,
            nm4[:N].reshape(N, 4, 1),
            em4.reshape(E, 4, 1),
            fm4[:N].reshape(N, 4, 128))
